# baseline (speedup 1.0000x reference)
import math

import jax
import jax.numpy as jnp
from jax import lax
from jax.experimental import pallas as pl
from jax.experimental.pallas import tpu as pltpu
from jax.experimental.pallas import tpu_sc as plsc

N = 10000
NPAD = 10240
D = 128
HD = 64
NC = 2
NS = 16
NW = NC * NS
LANES = 16
CB = 128
SLICE = NPAD // NS
ZERO16 = (LANES,)


def _sc_graph_kernel(x_hbm, e_hbm, t1p_out, dis_out,
                     ebufA, ebufB, colA, colB, sbufA, sbufB,
                     rowbufA, rowbufB, dis_vmem, slicebuf,
                     t1_sh, deg_sh, dis_sh,
                     esemA, esemB, gsemA, gsemB, ssemA, ssemB):
    nch = e_hbm.shape[1]
    ncht = 2 * nch
    nhalf = nch // 2
    c = lax.axis_index("c")
    s = lax.axis_index("s")
    wid_my = s * NC + c
    wid_ot = s * NC + (1 - c)
    base = s * SLICE

    def echunk(t):
        wid = jnp.where(t < nch, wid_my, wid_ot)
        j = jnp.where(t < nch, t, t - nch)
        return e_hbm.at[wid, j]

    scope_zero = jax.named_scope("ph_zero")
    scope_zero.__enter__()

    @plsc.parallel_loop(0, CB, 1, unroll=4)
    def _zrow(e):
        for k in range(D // LANES):
            rowbufA[e, pl.ds(k * LANES, LANES)] = jnp.zeros(ZERO16, jnp.float32)

    for i in range(SLICE // LANES):
        slicebuf[pl.ds(i * LANES, LANES)] = jnp.zeros(ZERO16, jnp.float32)
    pltpu.sync_copy(slicebuf, deg_sh.at[pl.ds(base, SLICE)])
    for r in range(SLICE // CB):
        pltpu.sync_copy(rowbufA, t1_sh.at[pl.ds(base + r * CB, CB)])
    plsc.subcore_barrier()
    scope_zero.__exit__(None, None, None)

    scope_deg = jax.named_scope("ph_deg")
    scope_deg.__enter__()
    pltpu.async_copy(echunk(0), ebufA, esemA)
    pltpu.async_copy(echunk(1), ebufB, esemB)

    def deg_section(t, ebuf, col, sbuf, esem, ssem, first):
        pltpu.make_async_copy(echunk(t), ebuf, esem).wait()
        if not first:
            pltpu.make_async_copy(sbuf, deg_sh.at[col], ssem).wait()
        for k in range(CB // LANES):
            col[pl.ds(k * LANES, LANES)] = ebuf[0, pl.ds(k * LANES, LANES)]
            sbuf[pl.ds(k * LANES, LANES)] = lax.bitcast_convert_type(
                ebuf[2, pl.ds(k * LANES, LANES)], jnp.float32)
        pltpu.async_copy(sbuf, deg_sh.at[col], ssem, add=True)

        @pl.when(t + 2 < ncht)
        def _():
            pltpu.async_copy(echunk(t + 2), ebuf, esem)

    def deg_body(t2, carry):
        deg_section(2 * t2, ebufA, colA, sbufA, esemA, ssemA, False)
        deg_section(2 * t2 + 1, ebufB, colB, sbufB, esemB, ssemB, False)
        return carry

    deg_section(0, ebufA, colA, sbufA, esemA, ssemA, True)
    deg_section(1, ebufB, colB, sbufB, esemB, ssemB, True)
    lax.fori_loop(1, nch, deg_body, 0)
    pltpu.make_async_copy(sbufA, deg_sh.at[colA], ssemA).wait()
    pltpu.make_async_copy(sbufB, deg_sh.at[colB], ssemB).wait()
    plsc.subcore_barrier()
    scope_deg.__exit__(None, None, None)
    scope_dis = jax.named_scope("ph_dis")
    scope_dis.__enter__()

    pltpu.sync_copy(deg_sh.at[pl.ds(base, SLICE)], slicebuf)
    for i in range(SLICE // LANES):
        dv = slicebuf[pl.ds(i * LANES, LANES)]
        bits = lax.bitcast_convert_type(dv, jnp.int32)
        y = lax.bitcast_convert_type(
            jnp.int32(0x5F3759DF) - jnp.right_shift(bits, 1), jnp.float32)
        for _ in range(3):
            y = y * (jnp.float32(1.5) - jnp.float32(0.5) * dv * y * y)
        y = jnp.where(dv > 0.0, y, jnp.float32(0.0))
        slicebuf[pl.ds(i * LANES, LANES)] = y
    pltpu.sync_copy(slicebuf, dis_sh.at[pl.ds(base, SLICE)])

    @pl.when(c == 0)
    def _():
        pltpu.sync_copy(slicebuf, dis_out.at[pl.ds(base, SLICE)])

    plsc.subcore_barrier()
    pltpu.sync_copy(dis_sh, dis_vmem)
    scope_dis.__exit__(None, None, None)
    scope_main = jax.named_scope("ph_main")
    scope_main.__enter__()

    def mchunk(j):
        return e_hbm.at[wid_my, j]

    def m_sect(j, ebuf, col, sbuf, rowbuf, esem, gsem, ssem):
        pltpu.make_async_copy(x_hbm.at[ebuf.at[0]], rowbuf, gsem).wait()
        for k in range(CB // LANES):
            rv = ebuf[0, pl.ds(k * LANES, LANES)]
            dv = plsc.load_gather(dis_vmem, [rv])
            wv = lax.bitcast_convert_type(
                ebuf[2, pl.ds(k * LANES, LANES)], jnp.float32)
            sbuf[pl.ds(k * LANES, LANES)] = dv * wv
            col[pl.ds(k * LANES, LANES)] = ebuf[1, pl.ds(k * LANES, LANES)]

        @plsc.parallel_loop(0, CB, 1, unroll=2)
        def _scale(e):
            sv = plsc.load_gather(sbuf, [jnp.zeros(ZERO16, jnp.int32) + e])
            for k in range(D // LANES):
                rowbuf[e, pl.ds(k * LANES, LANES)] = (
                    rowbuf[e, pl.ds(k * LANES, LANES)] * sv)

        pltpu.async_copy(rowbuf, t1_sh.at[col], ssem, add=True)

        @pl.when(j + 2 < nch)
        def _():
            pltpu.async_copy(mchunk(j + 2), ebuf, esem)

    def m_tail(j, ebuf, col, rowbuf, esem, gsem, ssem):
        pltpu.make_async_copy(rowbuf, t1_sh.at[col], ssem).wait()

        @pl.when(j + 2 < nch)
        def _():
            pltpu.make_async_copy(mchunk(j + 2), ebuf, esem).wait()
            pltpu.async_copy(x_hbm.at[ebuf.at[0]], rowbuf, gsem)

    def main_body(j2, carry):
        cA, cB = 2 * j2, 2 * j2 + 1
        m_sect(cA, ebufA, colA, sbufA, rowbufA, esemA, gsemA, ssemA)
        m_sect(cB, ebufB, colB, sbufB, rowbufB, esemB, gsemB, ssemB)
        m_tail(cA, ebufA, colA, rowbufA, esemA, gsemA, ssemA)
        m_tail(cB, ebufB, colB, rowbufB, esemB, gsemB, ssemB)
        return carry

    pltpu.async_copy(mchunk(0), ebufA, esemA)
    pltpu.async_copy(mchunk(1), ebufB, esemB)
    pltpu.make_async_copy(mchunk(0), ebufA, esemA).wait()
    pltpu.async_copy(x_hbm.at[ebufA.at[0]], rowbufA, gsemA)
    pltpu.make_async_copy(mchunk(1), ebufB, esemB).wait()
    pltpu.async_copy(x_hbm.at[ebufB.at[0]], rowbufB, gsemB)

    lax.fori_loop(0, nhalf, main_body, 0)
    plsc.subcore_barrier()
    scope_main.__exit__(None, None, None)
    with jax.named_scope("ph_out"):
        pltpu.sync_copy(t1_sh.at[pl.ds(base, SLICE)],
                        t1p_out.at[c, pl.ds(base, SLICE)])


def _tc_gru_kernel(x_ref, t1p_ref, dis_ref, w0_ref, w1_ref, bz_ref, bh_ref,
                   wl_ref, bl_ref, o_ref):
    dis = dis_ref[:N, :]
    t1 = (t1p_ref[0, :N, :] + t1p_ref[1, :N, :]) * (-dis)
    g0 = jnp.dot(x_ref[...], w0_ref[...], preferred_element_type=jnp.float32)
    g1 = jnp.dot(t1, w1_ref[...], preferred_element_type=jnp.float32)
    z = jax.nn.sigmoid(g0[:, :HD] + g1[:, :HD] + bz_ref[...])
    ht = jnp.tanh(g0[:, HD:] + g1[:, HD:] + bh_ref[...])
    hn = (1.0 - z) * ht
    o_ref[...] = jnp.sum(hn * wl_ref[...], axis=1) + bl_ref[0]


def kernel(x, edge_index, edge_weight, W_xz, b_xz, W_hz, b_hz, W_xr, b_xr,
           W_hr, b_hr, W_xh, b_xh, W_hh, b_hh, W_lin, b_lin):
    e = edge_index.shape[1]
    nch = math.ceil(e / (NW * CB))
    nch += nch % 2
    etot = NW * nch * CB
    pad = etot - e

    rowp = jnp.pad(edge_index[0], (0, pad)).reshape(NW, nch, CB)
    colp = jnp.pad(edge_index[1], (0, pad)).reshape(NW, nch, CB)
    wbits = lax.bitcast_convert_type(
        jnp.pad(edge_weight, (0, pad)), jnp.int32).reshape(NW, nch, CB)
    epack = jnp.stack([rowp, colp, wbits], axis=2)

    mesh = plsc.VectorSubcoreMesh(core_axis_name="c", subcore_axis_name="s")
    sck = pl.kernel(
        _sc_graph_kernel,
        out_type=(jax.ShapeDtypeStruct((NC, NPAD, D), jnp.float32),
                  jax.ShapeDtypeStruct((NPAD,), jnp.float32)),
        mesh=mesh,
        scratch_types=[
            pltpu.VMEM((3, CB), jnp.int32),
            pltpu.VMEM((3, CB), jnp.int32),
            pltpu.VMEM((CB,), jnp.int32),
            pltpu.VMEM((CB,), jnp.int32),
            pltpu.VMEM((CB,), jnp.float32),
            pltpu.VMEM((CB,), jnp.float32),
            pltpu.VMEM((CB, D), jnp.float32),
            pltpu.VMEM((CB, D), jnp.float32),
            pltpu.VMEM((NPAD,), jnp.float32),
            pltpu.VMEM((SLICE,), jnp.float32),
            pltpu.VMEM_SHARED((NPAD, D), jnp.float32),
            pltpu.VMEM_SHARED((NPAD,), jnp.float32),
            pltpu.VMEM_SHARED((NPAD,), jnp.float32),
            pltpu.SemaphoreType.DMA,
            pltpu.SemaphoreType.DMA,
            pltpu.SemaphoreType.DMA,
            pltpu.SemaphoreType.DMA,
            pltpu.SemaphoreType.DMA,
            pltpu.SemaphoreType.DMA,
        ],
        compiler_params=pltpu.CompilerParams(needs_layout_passes=False),
    )
    t1p, dis = sck(x, epack)

    w0 = jnp.concatenate([W_xz[0], W_xh[0]], axis=1)
    w1 = jnp.concatenate([W_xz[1], W_xh[1]], axis=1)
    bz = (b_xz + b_hz).reshape(1, HD)
    bh = (b_xh + b_hh).reshape(1, HD)
    wl = W_lin.reshape(1, HD)
    dis2 = dis.reshape(NPAD, 1)

    vspec = pl.BlockSpec(memory_space=pltpu.VMEM)
    out = pl.pallas_call(
        _tc_gru_kernel,
        out_shape=jax.ShapeDtypeStruct((N,), jnp.float32),
        in_specs=[vspec] * 8 + [pl.BlockSpec(memory_space=pltpu.SMEM)],
        out_specs=vspec,
    )(x, t1p, dis2, w0, w1, bz, bh, wl, b_lin)
    return out


# speedup vs baseline: 1.7158x; 1.7158x over previous
import math

import jax
import jax.numpy as jnp
from jax import lax
from jax.experimental import pallas as pl
from jax.experimental.pallas import tpu as pltpu
from jax.experimental.pallas import tpu_sc as plsc

N = 10000
NPAD = 10240
D = 128
HD = 64
NC = 2
NS = 16
NW = NC * NS
LANES = 16
CB = 128
SLICE = NPAD // NS
ZERO16 = (LANES,)
K0 = 118
K1 = 40


def _make_sc_kernel():
    kt = K0 + K1

    def body(x_hbm, e_hbm, t1p_out, dis_out,
             ebufA, ebufB, colA, colB, sbufA, sbufB,
             rowbufA, rowbufB, dis_vmem, slicebuf,
             t1_sh, deg_sh, dis_sh,
             esemA, esemB, gsemA, gsemB, ssemA, ssemB):
        c = lax.axis_index("c")
        s = lax.axis_index("s")
        wid_my = s * NC + c
        wid_ot = s * NC + (1 - c)
        cnt_my = jnp.where(c == 0, K0, K1)
        base = s * SLICE

        def echunk(t):
            wid = jnp.where(t < cnt_my, wid_my, wid_ot)
            j = jnp.where(t < cnt_my, t, t - cnt_my)
            return e_hbm.at[wid, j]

        def mchunk(j):
            return e_hbm.at[wid_my, j]

        @plsc.parallel_loop(0, CB, 1, unroll=4)
        def _zrow(e):
            for k in range(D // LANES):
                rowbufA[e, pl.ds(k * LANES, LANES)] = jnp.zeros(
                    ZERO16, jnp.float32)

        for i in range(SLICE // LANES):
            slicebuf[pl.ds(i * LANES, LANES)] = jnp.zeros(ZERO16, jnp.float32)
        pltpu.sync_copy(slicebuf, deg_sh.at[pl.ds(base, SLICE)])
        for r in range(SLICE // CB):
            pltpu.sync_copy(rowbufA, t1_sh.at[pl.ds(base + r * CB, CB)])
        plsc.subcore_barrier()

        pltpu.async_copy(echunk(0), ebufA, esemA)
        pltpu.async_copy(echunk(1), ebufB, esemB)

        def deg_section(t, ebuf, col, sbuf, esem, ssem, first):
            pltpu.make_async_copy(echunk(t), ebuf, esem).wait()
            if not first:
                pltpu.make_async_copy(sbuf, deg_sh.at[col], ssem).wait()
            for k in range(CB // LANES):
                col[pl.ds(k * LANES, LANES)] = ebuf[0, pl.ds(k * LANES, LANES)]
                sbuf[pl.ds(k * LANES, LANES)] = lax.bitcast_convert_type(
                    ebuf[2, pl.ds(k * LANES, LANES)], jnp.float32)
            pltpu.async_copy(sbuf, deg_sh.at[col], ssem, add=True)

            @pl.when(t + 2 < kt)
            def _():
                pltpu.async_copy(echunk(t + 2), ebuf, esem)

        def deg_body(t2, carry):
            deg_section(2 * t2, ebufA, colA, sbufA, esemA, ssemA, False)
            deg_section(2 * t2 + 1, ebufB, colB, sbufB, esemB, ssemB, False)
            return carry

        deg_section(0, ebufA, colA, sbufA, esemA, ssemA, True)
        deg_section(1, ebufB, colB, sbufB, esemB, ssemB, True)
        lax.fori_loop(1, kt // 2, deg_body, 0)
        pltpu.make_async_copy(sbufA, deg_sh.at[colA], ssemA).wait()
        pltpu.make_async_copy(sbufB, deg_sh.at[colB], ssemB).wait()
        plsc.subcore_barrier()

        pltpu.sync_copy(deg_sh.at[pl.ds(base, SLICE)], slicebuf)
        for i in range(SLICE // LANES):
            dv = slicebuf[pl.ds(i * LANES, LANES)]
            bits = lax.bitcast_convert_type(dv, jnp.int32)
            y = lax.bitcast_convert_type(
                jnp.int32(0x5F3759DF) - jnp.right_shift(bits, 1), jnp.float32)
            for _ in range(3):
                y = y * (jnp.float32(1.5) - jnp.float32(0.5) * dv * y * y)
            y = jnp.where(dv > 0.0, y, jnp.float32(0.0))
            slicebuf[pl.ds(i * LANES, LANES)] = y
        pltpu.sync_copy(slicebuf, dis_sh.at[pl.ds(base, SLICE)])

        @pl.when(c == 0)
        def _():
            pltpu.sync_copy(slicebuf, dis_out.at[pl.ds(base, SLICE)])

        plsc.subcore_barrier()
        pltpu.sync_copy(dis_sh, dis_vmem)

        def m_sect(j, nch, ebuf, col, sbuf, rowbuf, esem, gsem, ssem):
            pltpu.make_async_copy(x_hbm.at[ebuf.at[0]], rowbuf, gsem).wait()
            for k in range(CB // LANES):
                rv = ebuf[0, pl.ds(k * LANES, LANES)]
                dv = plsc.load_gather(dis_vmem, [rv])
                wv = lax.bitcast_convert_type(
                    ebuf[2, pl.ds(k * LANES, LANES)], jnp.float32)
                sbuf[pl.ds(k * LANES, LANES)] = dv * wv
                col[pl.ds(k * LANES, LANES)] = ebuf[1, pl.ds(k * LANES, LANES)]

            @plsc.parallel_loop(0, CB, 1, unroll=2)
            def _scale(e):
                sv = plsc.load_gather(sbuf, [jnp.zeros(ZERO16, jnp.int32) + e])
                for k in range(D // LANES):
                    rowbuf[e, pl.ds(k * LANES, LANES)] = (
                        rowbuf[e, pl.ds(k * LANES, LANES)] * sv)

            pltpu.async_copy(rowbuf, t1_sh.at[col], ssem, add=True)

            @pl.when(j + 2 < nch)
            def _():
                pltpu.async_copy(mchunk(j + 2), ebuf, esem)

        def m_tail(j, nch, ebuf, col, rowbuf, esem, gsem, ssem):
            pltpu.make_async_copy(rowbuf, t1_sh.at[col], ssem).wait()

            @pl.when(j + 2 < nch)
            def _():
                pltpu.make_async_copy(mchunk(j + 2), ebuf, esem).wait()
                pltpu.async_copy(x_hbm.at[ebuf.at[0]], rowbuf, gsem)

        def run_main(nch):
            def main_body(j2, carry):
                cA, cB = 2 * j2, 2 * j2 + 1
                m_sect(cA, nch, ebufA, colA, sbufA, rowbufA,
                       esemA, gsemA, ssemA)
                m_sect(cB, nch, ebufB, colB, sbufB, rowbufB,
                       esemB, gsemB, ssemB)
                m_tail(cA, nch, ebufA, colA, rowbufA, esemA, gsemA, ssemA)
                m_tail(cB, nch, ebufB, colB, rowbufB, esemB, gsemB, ssemB)
                return carry

            pltpu.async_copy(mchunk(0), ebufA, esemA)
            pltpu.async_copy(mchunk(1), ebufB, esemB)
            pltpu.make_async_copy(mchunk(0), ebufA, esemA).wait()
            pltpu.async_copy(x_hbm.at[ebufA.at[0]], rowbufA, gsemA)
            pltpu.make_async_copy(mchunk(1), ebufB, esemB).wait()
            pltpu.async_copy(x_hbm.at[ebufB.at[0]], rowbufB, gsemB)
            lax.fori_loop(0, nch // 2, main_body, 0)

        @pl.when(c == 0)
        def _():
            run_main(K0)

        @pl.when(c == 1)
        def _():
            run_main(K1)

        plsc.subcore_barrier()
        pltpu.sync_copy(t1_sh.at[pl.ds(base, SLICE)],
                        t1p_out.at[c, pl.ds(base, SLICE)])

    return body


def _tc_gru_kernel(x_ref, t1p_ref, dis_ref, w0_ref, w1_ref, bz_ref, bh_ref,
                   wl_ref, bl_ref, o_ref):
    dis = dis_ref[:N, :]
    t1 = (t1p_ref[0, :N, :] + t1p_ref[1, :N, :]) * (-dis)
    g0 = jnp.dot(x_ref[...], w0_ref[...], preferred_element_type=jnp.float32)
    g1 = jnp.dot(t1, w1_ref[...], preferred_element_type=jnp.float32)
    z = jax.nn.sigmoid(g0[:, :HD] + g1[:, :HD] + bz_ref[...])
    ht = jnp.tanh(g0[:, HD:] + g1[:, HD:] + bh_ref[...])
    hn = (1.0 - z) * ht
    o_ref[...] = jnp.sum(hn * wl_ref[...], axis=1) + bl_ref[0]


def kernel(x, edge_index, edge_weight, W_xz, b_xz, W_hz, b_hz, W_xr, b_xr,
           W_hr, b_hr, W_xh, b_xh, W_hh, b_hh, W_lin, b_lin):
    e = edge_index.shape[1]
    ncap = NS * (K0 + K1) * CB
    pad = ncap - e

    def chunks(a):
        return jnp.pad(a, (0, pad)).reshape(NS * (K0 + K1), CB)

    rowp = chunks(edge_index[0])
    colp = chunks(edge_index[1])
    wbits = chunks(lax.bitcast_convert_type(edge_weight, jnp.int32))
    flat = jnp.stack([rowp, colp, wbits], axis=1)
    e0 = flat[:NS * K0].reshape(NS, 1, K0, 3, CB)
    e1 = jnp.pad(flat[NS * K0:].reshape(NS, 1, K1, 3, CB),
                 ((0, 0), (0, 0), (0, K0 - K1), (0, 0), (0, 0)))
    epack = jnp.concatenate([e0, e1], axis=1).reshape(NW, K0, 3, CB)

    mesh = plsc.VectorSubcoreMesh(core_axis_name="c", subcore_axis_name="s")
    sck = pl.kernel(
        _make_sc_kernel(),
        out_type=(jax.ShapeDtypeStruct((NC, NPAD, D), jnp.float32),
                  jax.ShapeDtypeStruct((NPAD,), jnp.float32)),
        mesh=mesh,
        scratch_types=[
            pltpu.VMEM((3, CB), jnp.int32),
            pltpu.VMEM((3, CB), jnp.int32),
            pltpu.VMEM((CB,), jnp.int32),
            pltpu.VMEM((CB,), jnp.int32),
            pltpu.VMEM((CB,), jnp.float32),
            pltpu.VMEM((CB,), jnp.float32),
            pltpu.VMEM((CB, D), jnp.float32),
            pltpu.VMEM((CB, D), jnp.float32),
            pltpu.VMEM((NPAD,), jnp.float32),
            pltpu.VMEM((SLICE,), jnp.float32),
            pltpu.VMEM_SHARED((NPAD, D), jnp.float32),
            pltpu.VMEM_SHARED((NPAD,), jnp.float32),
            pltpu.VMEM_SHARED((NPAD,), jnp.float32),
            pltpu.SemaphoreType.DMA,
            pltpu.SemaphoreType.DMA,
            pltpu.SemaphoreType.DMA,
            pltpu.SemaphoreType.DMA,
            pltpu.SemaphoreType.DMA,
            pltpu.SemaphoreType.DMA,
        ],
        compiler_params=pltpu.CompilerParams(needs_layout_passes=False),
    )
    t1p, dis = sck(x, epack)

    w0 = jnp.concatenate([W_xz[0], W_xh[0]], axis=1)
    w1 = jnp.concatenate([W_xz[1], W_xh[1]], axis=1)
    bz = (b_xz + b_hz).reshape(1, HD)
    bh = (b_xh + b_hh).reshape(1, HD)
    wl = W_lin.reshape(1, HD)
    dis2 = dis.reshape(NPAD, 1)

    vspec = pl.BlockSpec(memory_space=pltpu.VMEM)
    out = pl.pallas_call(
        _tc_gru_kernel,
        out_shape=jax.ShapeDtypeStruct((N,), jnp.float32),
        in_specs=[vspec] * 8 + [pl.BlockSpec(memory_space=pltpu.SMEM)],
        out_specs=vspec,
    )(x, t1p, dis2, w0, w1, bz, bh, wl, b_lin)
    return out


# speedup vs baseline: 1.7413x; 1.0149x over previous
import math

import jax
import jax.numpy as jnp
from jax import lax
from jax.experimental import pallas as pl
from jax.experimental.pallas import tpu as pltpu
from jax.experimental.pallas import tpu_sc as plsc

N = 10000
NPAD = 10240
D = 128
HD = 64
NC = 2
NS = 16
NW = NC * NS
LANES = 16
CB = 128
SLICE = NPAD // NS
ZERO16 = (LANES,)
K0 = 122
K1 = 36


def _make_sc_kernel():
    kt = K0 + K1

    def body(x_hbm, e_hbm, t1p_out, dis_out,
             ebufA, ebufB, colA, colB, sbufA, sbufB,
             rowbufA, rowbufB, dis_vmem, slicebuf,
             t1_sh, deg_sh, dis_sh,
             esemA, esemB, gsemA, gsemB, ssemA, ssemB):
        c = lax.axis_index("c")
        s = lax.axis_index("s")
        wid_my = s * NC + c
        wid_ot = s * NC + (1 - c)
        cnt_my = jnp.where(c == 0, K0, K1)
        base = s * SLICE

        def echunk(t):
            wid = jnp.where(t < cnt_my, wid_my, wid_ot)
            j = jnp.where(t < cnt_my, t, t - cnt_my)
            return e_hbm.at[wid, j]

        def mchunk(j):
            return e_hbm.at[wid_my, j]

        @plsc.parallel_loop(0, CB, 1, unroll=4)
        def _zrow(e):
            for k in range(D // LANES):
                rowbufA[e, pl.ds(k * LANES, LANES)] = jnp.zeros(
                    ZERO16, jnp.float32)

        for i in range(SLICE // LANES):
            slicebuf[pl.ds(i * LANES, LANES)] = jnp.zeros(ZERO16, jnp.float32)
        pltpu.sync_copy(slicebuf, deg_sh.at[pl.ds(base, SLICE)])
        for r in range(SLICE // CB):
            pltpu.sync_copy(rowbufA, t1_sh.at[pl.ds(base + r * CB, CB)])
        plsc.subcore_barrier()

        pltpu.async_copy(echunk(0), ebufA, esemA)
        pltpu.async_copy(echunk(1), ebufB, esemB)

        def deg_section(t, ebuf, col, sbuf, esem, ssem, first):
            pltpu.make_async_copy(echunk(t), ebuf, esem).wait()
            if not first:
                pltpu.make_async_copy(sbuf, deg_sh.at[col], ssem).wait()
            for k in range(CB // LANES):
                col[pl.ds(k * LANES, LANES)] = ebuf[0, pl.ds(k * LANES, LANES)]
                sbuf[pl.ds(k * LANES, LANES)] = lax.bitcast_convert_type(
                    ebuf[2, pl.ds(k * LANES, LANES)], jnp.float32)
            pltpu.async_copy(sbuf, deg_sh.at[col], ssem, add=True)

            @pl.when(t + 2 < kt)
            def _():
                pltpu.async_copy(echunk(t + 2), ebuf, esem)

        def deg_body(t2, carry):
            deg_section(2 * t2, ebufA, colA, sbufA, esemA, ssemA, False)
            deg_section(2 * t2 + 1, ebufB, colB, sbufB, esemB, ssemB, False)
            return carry

        deg_section(0, ebufA, colA, sbufA, esemA, ssemA, True)
        deg_section(1, ebufB, colB, sbufB, esemB, ssemB, True)
        lax.fori_loop(1, kt // 2, deg_body, 0)
        pltpu.make_async_copy(sbufA, deg_sh.at[colA], ssemA).wait()
        pltpu.make_async_copy(sbufB, deg_sh.at[colB], ssemB).wait()
        plsc.subcore_barrier()

        pltpu.sync_copy(deg_sh.at[pl.ds(base, SLICE)], slicebuf)
        for i in range(SLICE // LANES):
            dv = slicebuf[pl.ds(i * LANES, LANES)]
            bits = lax.bitcast_convert_type(dv, jnp.int32)
            y = lax.bitcast_convert_type(
                jnp.int32(0x5F3759DF) - jnp.right_shift(bits, 1), jnp.float32)
            for _ in range(3):
                y = y * (jnp.float32(1.5) - jnp.float32(0.5) * dv * y * y)
            y = jnp.where(dv > 0.0, y, jnp.float32(0.0))
            slicebuf[pl.ds(i * LANES, LANES)] = y
        pltpu.sync_copy(slicebuf, dis_sh.at[pl.ds(base, SLICE)])

        @pl.when(c == 0)
        def _():
            pltpu.sync_copy(slicebuf, dis_out.at[pl.ds(base, SLICE)])

        plsc.subcore_barrier()
        pltpu.sync_copy(dis_sh, dis_vmem)

        def m_sect(j, nch, ebuf, col, sbuf, rowbuf, esem, gsem, ssem):
            pltpu.make_async_copy(x_hbm.at[ebuf.at[0]], rowbuf, gsem).wait()
            for k in range(CB // LANES):
                rv = ebuf[0, pl.ds(k * LANES, LANES)]
                dv = plsc.load_gather(dis_vmem, [rv])
                wv = lax.bitcast_convert_type(
                    ebuf[2, pl.ds(k * LANES, LANES)], jnp.float32)
                sbuf[pl.ds(k * LANES, LANES)] = dv * wv
                col[pl.ds(k * LANES, LANES)] = ebuf[1, pl.ds(k * LANES, LANES)]

            @plsc.parallel_loop(0, CB, 1, unroll=2)
            def _scale(e):
                sv = plsc.load_gather(sbuf, [jnp.zeros(ZERO16, jnp.int32) + e])
                for k in range(D // LANES):
                    rowbuf[e, pl.ds(k * LANES, LANES)] = (
                        rowbuf[e, pl.ds(k * LANES, LANES)] * sv)

            pltpu.async_copy(rowbuf, t1_sh.at[col], ssem, add=True)

            @pl.when(j + 2 < nch)
            def _():
                pltpu.async_copy(mchunk(j + 2), ebuf, esem)

        def m_tail(j, nch, ebuf, col, rowbuf, esem, gsem, ssem):
            pltpu.make_async_copy(rowbuf, t1_sh.at[col], ssem).wait()

            @pl.when(j + 2 < nch)
            def _():
                pltpu.make_async_copy(mchunk(j + 2), ebuf, esem).wait()
                pltpu.async_copy(x_hbm.at[ebuf.at[0]], rowbuf, gsem)

        def run_main(nch):
            def main_body(j2, carry):
                cA, cB = 2 * j2, 2 * j2 + 1
                m_sect(cA, nch, ebufA, colA, sbufA, rowbufA,
                       esemA, gsemA, ssemA)
                m_sect(cB, nch, ebufB, colB, sbufB, rowbufB,
                       esemB, gsemB, ssemB)
                m_tail(cA, nch, ebufA, colA, rowbufA, esemA, gsemA, ssemA)
                m_tail(cB, nch, ebufB, colB, rowbufB, esemB, gsemB, ssemB)
                return carry

            pltpu.async_copy(mchunk(0), ebufA, esemA)
            pltpu.async_copy(mchunk(1), ebufB, esemB)
            pltpu.make_async_copy(mchunk(0), ebufA, esemA).wait()
            pltpu.async_copy(x_hbm.at[ebufA.at[0]], rowbufA, gsemA)
            pltpu.make_async_copy(mchunk(1), ebufB, esemB).wait()
            pltpu.async_copy(x_hbm.at[ebufB.at[0]], rowbufB, gsemB)
            lax.fori_loop(0, nch // 2, main_body, 0)

        @pl.when(c == 0)
        def _():
            run_main(K0)

        @pl.when(c == 1)
        def _():
            run_main(K1)

        plsc.subcore_barrier()
        pltpu.sync_copy(t1_sh.at[pl.ds(base, SLICE)],
                        t1p_out.at[c, pl.ds(base, SLICE)])

    return body


def _tc_gru_kernel(x_ref, t1p_ref, dis_ref, w0_ref, w1_ref, bz_ref, bh_ref,
                   wl_ref, bl_ref, o_ref):
    dis = dis_ref[:N, :]
    t1 = (t1p_ref[0, :N, :] + t1p_ref[1, :N, :]) * (-dis)
    g0 = jnp.dot(x_ref[...], w0_ref[...], preferred_element_type=jnp.float32)
    g1 = jnp.dot(t1, w1_ref[...], preferred_element_type=jnp.float32)
    z = jax.nn.sigmoid(g0[:, :HD] + g1[:, :HD] + bz_ref[...])
    ht = jnp.tanh(g0[:, HD:] + g1[:, HD:] + bh_ref[...])
    hn = (1.0 - z) * ht
    o_ref[...] = jnp.sum(hn * wl_ref[...], axis=1) + bl_ref[0]


def kernel(x, edge_index, edge_weight, W_xz, b_xz, W_hz, b_hz, W_xr, b_xr,
           W_hr, b_hr, W_xh, b_xh, W_hh, b_hh, W_lin, b_lin):
    e = edge_index.shape[1]
    ncap = NS * (K0 + K1) * CB
    pad = ncap - e

    def chunks(a):
        return jnp.pad(a, (0, pad)).reshape(NS * (K0 + K1), CB)

    rowp = chunks(edge_index[0])
    colp = chunks(edge_index[1])
    wbits = chunks(lax.bitcast_convert_type(edge_weight, jnp.int32))
    flat = jnp.stack([rowp, colp, wbits], axis=1)
    e0 = flat[:NS * K0].reshape(NS, 1, K0, 3, CB)
    e1 = jnp.pad(flat[NS * K0:].reshape(NS, 1, K1, 3, CB),
                 ((0, 0), (0, 0), (0, K0 - K1), (0, 0), (0, 0)))
    epack = jnp.concatenate([e0, e1], axis=1).reshape(NW, K0, 3, CB)

    mesh = plsc.VectorSubcoreMesh(core_axis_name="c", subcore_axis_name="s")
    sck = pl.kernel(
        _make_sc_kernel(),
        out_type=(jax.ShapeDtypeStruct((NC, NPAD, D), jnp.float32),
                  jax.ShapeDtypeStruct((NPAD,), jnp.float32)),
        mesh=mesh,
        scratch_types=[
            pltpu.VMEM((3, CB), jnp.int32),
            pltpu.VMEM((3, CB), jnp.int32),
            pltpu.VMEM((CB,), jnp.int32),
            pltpu.VMEM((CB,), jnp.int32),
            pltpu.VMEM((CB,), jnp.float32),
            pltpu.VMEM((CB,), jnp.float32),
            pltpu.VMEM((CB, D), jnp.float32),
            pltpu.VMEM((CB, D), jnp.float32),
            pltpu.VMEM((NPAD,), jnp.float32),
            pltpu.VMEM((SLICE,), jnp.float32),
            pltpu.VMEM_SHARED((NPAD, D), jnp.float32),
            pltpu.VMEM_SHARED((NPAD,), jnp.float32),
            pltpu.VMEM_SHARED((NPAD,), jnp.float32),
            pltpu.SemaphoreType.DMA,
            pltpu.SemaphoreType.DMA,
            pltpu.SemaphoreType.DMA,
            pltpu.SemaphoreType.DMA,
            pltpu.SemaphoreType.DMA,
            pltpu.SemaphoreType.DMA,
        ],
        compiler_params=pltpu.CompilerParams(needs_layout_passes=False),
    )
    t1p, dis = sck(x, epack)

    w0 = jnp.concatenate([W_xz[0], W_xh[0]], axis=1)
    w1 = jnp.concatenate([W_xz[1], W_xh[1]], axis=1)
    bz = (b_xz + b_hz).reshape(1, HD)
    bh = (b_xh + b_hh).reshape(1, HD)
    wl = W_lin.reshape(1, HD)
    dis2 = dis.reshape(NPAD, 1)

    vspec = pl.BlockSpec(memory_space=pltpu.VMEM)
    out = pl.pallas_call(
        _tc_gru_kernel,
        out_shape=jax.ShapeDtypeStruct((N,), jnp.float32),
        in_specs=[vspec] * 8 + [pl.BlockSpec(memory_space=pltpu.SMEM)],
        out_specs=vspec,
    )(x, t1p, dis2, w0, w1, bz, bh, wl, b_lin)
    return out
